# baseline (speedup 1.0000x reference)
import functools

import jax
import jax.numpy as jnp
from jax import lax
from jax.experimental import pallas as pl
from jax.experimental.pallas import tpu as pltpu
from jax.experimental.pallas import tpu_sc as plsc

_NC = 2
_NS = 16
_NW = _NC * _NS
_PITCH = 129


def _make_gather(B, H, V, D):
    n_hg = H // 8
    n_bb = B // 128
    units_per_w = (n_hg * n_bb) // _NW
    mesh = plsc.VectorSubcoreMesh(core_axis_name="c", subcore_axis_name="s")

    @functools.partial(
        pl.kernel,
        mesh=mesh,
        out_type=jax.ShapeDtypeStruct((H, D, B), jnp.float32),
        compiler_params=pltpu.CompilerParams(needs_layout_passes=False),
        scratch_types=[
            pltpu.VMEM((8, 128), jnp.int32),
            pltpu.VMEM((128,), jnp.int32),
            pltpu.VMEM((128,), jnp.int32),
            pltpu.VMEM((128, 128), jnp.float32),
            pltpu.VMEM((128, 128), jnp.float32),
            pltpu.VMEM((D * _PITCH + 96,), jnp.float32),
            pltpu.VMEM((D, 128), jnp.float32),
            pltpu.VMEM((D, 128), jnp.float32),
            pltpu.SemaphoreType.DMA,
            pltpu.SemaphoreType.DMA,
            pltpu.SemaphoreType.DMA,
        ],
    )
    def k(xt_hbm, tab_hbm, out_hbm, idx_v, lin0, lin1, g0, g1, sb, o0, o1,
          isem, gsem, osem):
        wid = lax.axis_index("s") * _NC + lax.axis_index("c")
        lin_bufs = (lin0, lin1)
        g_bufs = (g0, g1)
        o_bufs = (o0, o1)

        iota16 = lax.iota(jnp.int32, 16)
        pre = tuple(
            (iota16 + dj * 16) * _PITCH for dj in range(D // 16)
        )

        def unit_body(u, carry):
            uid = wid * units_per_w + u
            hg = uid // n_bb
            bb = uid % n_bb

            icp = pltpu.make_async_copy(
                xt_hbm.at[pl.ds(hg * 8, 8), pl.ds(bb * 128, 128)],
                idx_v, isem,
            )
            icp.start()
            icp.wait()

            def fill_lines(s, hh):
                for j in range(8):
                    lin_bufs[s][pl.ds(j * 16, 16)] = idx_v[
                        hh, pl.ds(j * 16, 16)
                    ]

            def gather_cp(s):
                return pltpu.make_async_copy(
                    tab_hbm.at[lin_bufs[s]], g_bufs[s], gsem
                )

            def store_cp(s, hh):
                return pltpu.make_async_copy(
                    o_bufs[s],
                    out_hbm.at[hg * 8 + hh, :, pl.ds(bb * 128, 128)],
                    osem,
                )

            def transpose(s):
                def bgrp(bg, c, s=s):
                    base = bg * 32
                    sbs = sb.at[pl.ds(base, D * _PITCH)]
                    for k8 in range(32):
                        b = base + k8
                        for dj in range(D // 16):
                            v = g_bufs[s][b, pl.ds(dj * 16, 16)]
                            plsc.store_scatter(sbs, [pre[dj] + k8], v)
                    return c

                lax.fori_loop(0, 4, bgrp, 0)

                def dgrp(dg, c, s=s):
                    for k2 in range(16):
                        d = dg * 16 + k2
                        for j in range(8):
                            o_bufs[s][d, pl.ds(j * 16, 16)] = sb[
                                pl.ds(d * _PITCH + j * 16, 16)
                            ]
                    return c

                lax.fori_loop(0, D // 16, dgrp, 0)

            fill_lines(0, 0)
            gather_cp(0).start()
            for hh in range(8):
                s = hh % 2
                o = 1 - s
                if hh < 7:
                    fill_lines(o, hh + 1)
                gather_cp(s).wait()
                if hh < 7:
                    gather_cp(o).start()
                if hh >= 2:
                    store_cp(s, hh - 2).wait()
                transpose(s)
                store_cp(s, hh).start()
            store_cp(0, 6).wait()
            store_cp(1, 7).wait()
            return carry

        lax.fori_loop(0, units_per_w, unit_body, 0)

    return k


def kernel(x, table):
    Bb, H = x.shape
    V, D = table.shape
    xt = x.T.astype(jnp.int32)
    tabp = jnp.concatenate(
        [table, jnp.zeros((V, 128 - D), jnp.float32)], axis=1
    )
    out_t = _make_gather(Bb, H, V, D)(xt, tabp)
    return jnp.transpose(out_t, (2, 0, 1))


# speedup vs baseline: 2.0819x; 2.0819x over previous
import functools

import jax
import jax.numpy as jnp
from jax import lax
from jax.experimental import pallas as pl
from jax.experimental.pallas import tpu as pltpu
from jax.experimental.pallas import tpu_sc as plsc

_NC = 2
_NS = 16
_NW = _NC * _NS
_CHUNK = 800


def _make_gather(B, V, D):
    b_per_w = B // _NW
    nchunks = b_per_w // _CHUNK
    npairs = nchunks // 2
    mesh = plsc.VectorSubcoreMesh(core_axis_name="c", subcore_axis_name="s")

    @functools.partial(
        pl.kernel,
        mesh=mesh,
        out_type=jax.ShapeDtypeStruct((B, 2 * D), jnp.float32),
        compiler_params=pltpu.CompilerParams(use_tc_tiling_on_sc=False),
        scratch_types=[
            pltpu.VMEM((_CHUNK,), jnp.int32),
            pltpu.VMEM((_CHUNK,), jnp.int32),
            pltpu.VMEM((_CHUNK, D), jnp.float32),
            pltpu.VMEM((_CHUNK, D), jnp.float32),
            pltpu.SemaphoreType.DMA,
            pltpu.SemaphoreType.DMA,
            pltpu.SemaphoreType.DMA,
        ],
    )
    def k(idx_hbm, table_hbm, out_hbm, idx_v0, idx_v1, rows_v0, rows_v1,
          isem, gsem, osem):
        wid = lax.axis_index("s") * _NC + lax.axis_index("c")
        base = wid * b_per_w
        idx_bufs = (idx_v0, idx_v1)
        row_bufs = (rows_v0, rows_v1)
        n = nchunks

        def idx_copy(c, s):
            return pltpu.make_async_copy(
                idx_hbm.at[pl.ds(base + c * _CHUNK, _CHUNK)],
                idx_bufs[s],
                isem,
            )

        def gather_copy(s):
            return pltpu.make_async_copy(
                table_hbm.at[idx_bufs[s]], row_bufs[s], gsem
            )

        def store_copy(c, s):
            return pltpu.make_async_copy(
                row_bufs[s],
                out_hbm.at[pl.ds(base + c * _CHUNK, _CHUNK), pl.ds(0, D)],
                osem,
            )

        idx_copy(0, 0).start()
        idx_copy(0, 0).wait()
        gather_copy(0).start()
        idx_copy(1, 1).start()

        def pair_body(g, carry):
            for sbuf in range(2):
                c = g * 2 + sbuf
                obuf = 1 - sbuf

                @pl.when(c < n - 1)
                def _():
                    @pl.when(c >= 1)
                    def _():
                        store_copy(c - 1, obuf).wait()

                    idx_copy(c + 1, obuf).wait()
                    gather_copy(obuf).start()

                gather_copy(sbuf).wait()
                store_copy(c, sbuf).start()

                @pl.when(c < n - 2)
                def _():
                    idx_copy(c + 2, sbuf).start()
            return carry

        lax.fori_loop(0, npairs, pair_body, 0)

        store_copy(n - 2, (n - 2) % 2).wait()
        store_copy(n - 1, (n - 1) % 2).wait()

    return k


def kernel(x, table):
    Bb, H = x.shape
    V, D = table.shape
    B = Bb * H
    idx_flat = x.reshape(B).astype(jnp.int32)
    wide = _make_gather(B, V, D)(idx_flat, table)
    return wide[:, :D].reshape(Bb, H, D)
